# baseline (speedup 1.0000x reference)
import functools

import jax
import jax.numpy as jnp
from jax import lax
from jax.experimental import pallas as pl
from jax.experimental.pallas import tpu as pltpu
from jax.experimental.pallas import tpu_sc as plsc

N = 10000
E = 320000
D = 128
DE = 4

NC = 2
NS = 16
NW = NC * NS
CHUNK = 128
NCHUNK = 80
GRP = 8
NGRP = NCHUNK // GRP
EPW = CHUNK * NCHUNK
EPAD = NW * EPW
NPAD = 10240
ROWS_PT = NPAD // NS
ZROWS = 8


def _ef_body(ea_ref, we_ref, out_ref):
    out_ref[...] = jnp.dot(ea_ref[...], we_ref[...],
                           preferred_element_type=jnp.float32)


def _edge_feat(edge_attr, We):
    B = 4096
    return pl.pallas_call(
        _ef_body,
        grid=(EPAD // B,),
        in_specs=[pl.BlockSpec((B, DE), lambda i: (i, 0)),
                  pl.BlockSpec((DE, D), lambda i: (0, 0))],
        out_specs=pl.BlockSpec((B, D), lambda i: (i, 0)),
        out_shape=jax.ShapeDtypeStruct((EPAD, D), jnp.float32),
    )(edge_attr, We)


def _hmm_body(x_ref, w_ref, b_ref, out_ref):
    out_ref[...] = jnp.dot(x_ref[...], w_ref[...],
                           preferred_element_type=jnp.float32) + b_ref[...]


def _hmm(x, W, b2):
    B = 2000
    return pl.pallas_call(
        _hmm_body,
        grid=(N // B,),
        in_specs=[pl.BlockSpec((B, D), lambda i: (i, 0)),
                  pl.BlockSpec((D, D), lambda i: (0, 0)),
                  pl.BlockSpec((1, D), lambda i: (0, 0))],
        out_specs=pl.BlockSpec((B, D), lambda i: (i, 0)),
        out_shape=jax.ShapeDtypeStruct((N, D), jnp.float32),
    )(x, W, b2)


def _upd_common(x_ref, h_ref, a0_ref, a1_ref, g_ref, be_ref, rm_ref, rv_ref):
    u = h_ref[...] + a0_ref[...] + a1_ref[...]
    scale = g_ref[...] * lax.rsqrt(rv_ref[...] + 1e-5)
    u = (u - rm_ref[...]) * scale + be_ref[...]
    u = jnp.maximum(u, 0.0)
    return 0.5 * x_ref[...] + 0.5 * u


def _updmm_body(x_ref, h_ref, a0_ref, a1_ref, g_ref, be_ref, rm_ref, rv_ref,
                w_ref, b_ref, xo_ref, ho_ref):
    xn = _upd_common(x_ref, h_ref, a0_ref, a1_ref, g_ref, be_ref, rm_ref, rv_ref)
    xo_ref[...] = xn
    ho_ref[...] = jnp.dot(xn, w_ref[...],
                          preferred_element_type=jnp.float32) + b_ref[...]


def _upd_last_body(x_ref, h_ref, a0_ref, a1_ref, g_ref, be_ref, rm_ref, rv_ref,
                   xo_ref):
    xo_ref[...] = _upd_common(x_ref, h_ref, a0_ref, a1_ref,
                              g_ref, be_ref, rm_ref, rv_ref)


def _update_mm(x, h, a0, a1, g2, be2, rm2, rv2, W, b2):
    B = 2000
    row = lambda i: (i, 0)
    fixed = lambda i: (0, 0)
    return pl.pallas_call(
        _updmm_body,
        grid=(N // B,),
        in_specs=[pl.BlockSpec((B, D), row), pl.BlockSpec((B, D), row),
                  pl.BlockSpec((B, D), row), pl.BlockSpec((B, D), row),
                  pl.BlockSpec((1, D), fixed), pl.BlockSpec((1, D), fixed),
                  pl.BlockSpec((1, D), fixed), pl.BlockSpec((1, D), fixed),
                  pl.BlockSpec((D, D), fixed), pl.BlockSpec((1, D), fixed)],
        out_specs=[pl.BlockSpec((B, D), row), pl.BlockSpec((B, D), row)],
        out_shape=[jax.ShapeDtypeStruct((N, D), jnp.float32),
                   jax.ShapeDtypeStruct((N, D), jnp.float32)],
    )(x, h, a0, a1, g2, be2, rm2, rv2, W, b2)


def _update_last(x, h, a0, a1, g2, be2, rm2, rv2):
    B = 2000
    row = lambda i: (i, 0)
    fixed = lambda i: (0, 0)
    return pl.pallas_call(
        _upd_last_body,
        grid=(N // B,),
        in_specs=[pl.BlockSpec((B, D), row), pl.BlockSpec((B, D), row),
                  pl.BlockSpec((B, D), row), pl.BlockSpec((B, D), row),
                  pl.BlockSpec((1, D), fixed), pl.BlockSpec((1, D), fixed),
                  pl.BlockSpec((1, D), fixed), pl.BlockSpec((1, D), fixed)],
        out_specs=pl.BlockSpec((B, D), row),
        out_shape=jax.ShapeDtypeStruct((N, D), jnp.float32),
    )(x, h, a0, a1, g2, be2, rm2, rv2)


def _edge_pass_body(h_hbm, src_hbm, dst_hbm, e_hbm, out_hbm,
                    gsb, gdb, hrows, erows, zbuf, agg_sh, sem_h, sem_e):
    c = lax.axis_index("c")
    s = lax.axis_index("s")
    wid = s * NC + c
    ebase = wid * EPW

    def zrow(j, _):
        for t in range(D // 16):
            zbuf[j, pl.ds(t * 16, 16)] = jnp.zeros((16,), jnp.float32)
        return 0
    lax.fori_loop(0, ZROWS, zrow, 0)
    for k in range(ROWS_PT // ZROWS):
        pltpu.sync_copy(zbuf, agg_sh.at[pl.ds(s * ROWS_PT + k * ZROWS, ZROWS)])
    plsc.subcore_barrier()

    def compute():
        def row(j, _):
            for t in range(D // 16):
                sl = pl.ds(t * 16, 16)
                hrows[j, sl] = jnp.maximum(hrows[j, sl] + erows[j, sl], 0.0)
            return 0
        lax.fori_loop(0, CHUNK, row, 0)

    for g in range(NGRP):
        pltpu.sync_copy(src_hbm.at[wid * NGRP + g], gsb)
        pltpu.sync_copy(dst_hbm.at[wid * NGRP + g], gdb)
        for j in range(GRP):
            i = GRP * g + j
            ch = pltpu.async_copy(h_hbm.at[gsb.at[j]], hrows, sem_h)
            ce = pltpu.async_copy(e_hbm.at[pl.ds((ebase + i * CHUNK), CHUNK)],
                                  erows, sem_e)
            ch.wait()
            ce.wait()
            compute()
            pltpu.sync_copy(hrows, agg_sh.at[gdb.at[j]], add=True)
    plsc.subcore_barrier()

    pltpu.sync_copy(agg_sh.at[pl.ds(s * ROWS_PT, ROWS_PT)],
                    out_hbm.at[c, pl.ds(s * ROWS_PT, ROWS_PT)])


_edge_pass = functools.partial(
    pl.kernel,
    out_type=jax.ShapeDtypeStruct((NC, NPAD, D), jnp.float32),
    mesh=plsc.VectorSubcoreMesh(core_axis_name="c", subcore_axis_name="s"),
    scratch_types=[
        pltpu.VMEM((GRP, CHUNK), jnp.int32),
        pltpu.VMEM((GRP, CHUNK), jnp.int32),
        pltpu.VMEM((CHUNK, D), jnp.float32),
        pltpu.VMEM((CHUNK, D), jnp.float32),
        pltpu.VMEM((ZROWS, D), jnp.float32),
        pltpu.VMEM_SHARED((NPAD, D), jnp.float32),
        pltpu.SemaphoreType.DMA,
        pltpu.SemaphoreType.DMA,
    ],
)(_edge_pass_body)


def kernel(x, edge_index, edge_attr, batch, W, b, We, gamma, beta,
           run_mean, run_var):
    pad = EPAD - E
    src = jnp.concatenate([edge_index[0],
                           jnp.zeros((pad,), edge_index.dtype)])
    dst = jnp.concatenate([edge_index[1],
                           N + (jnp.arange(pad, dtype=edge_index.dtype)
                                % (NPAD - N))])
    edge_attr = jnp.concatenate(
        [edge_attr, jnp.zeros((pad, DE), edge_attr.dtype)])
    src = src.reshape(NW * NGRP, GRP, CHUNK)
    dst = dst.reshape(NW * NGRP, GRP, CHUNK)
    b2 = b.reshape(1, D)
    g2 = gamma.reshape(1, D)
    be2 = beta.reshape(1, D)
    rm2 = run_mean.reshape(1, D)
    rv2 = run_var.reshape(1, D)

    e = _edge_feat(edge_attr, We)
    h = _hmm(x, W, b2)
    for i in range(4):
        aggs = _edge_pass(h, src, dst, e)
        a0 = aggs[0, :N]
        a1 = aggs[1, :N]
        if i < 3:
            x, h = _update_mm(x, h, a0, a1, g2, be2, rm2, rv2, W, b2)
        else:
            x = _update_last(x, h, a0, a1, g2, be2, rm2, rv2)
    return x


# speedup vs baseline: 1.5517x; 1.5517x over previous
import functools

import jax
import jax.numpy as jnp
from jax import lax
from jax.experimental import pallas as pl
from jax.experimental.pallas import tpu as pltpu
from jax.experimental.pallas import tpu_sc as plsc

N = 10000
E = 320000
D = 128
DE = 4

NC = 2
NS = 16
NW = NC * NS
CHUNK = 80
EPW = E // NW
NCHUNK = EPW // CHUNK
NPAD = 10240
ROWS_PT = NPAD // NS
ZROWS = 128


def _ef_body(ea_ref, we_ref, out_ref):
    out_ref[...] = jnp.dot(ea_ref[...], we_ref[...],
                           preferred_element_type=jnp.float32)


def _edge_feat(edge_attr, We):
    B = 4000
    return pl.pallas_call(
        _ef_body,
        grid=(E // B,),
        in_specs=[pl.BlockSpec((B, DE), lambda i: (i, 0)),
                  pl.BlockSpec((DE, D), lambda i: (0, 0))],
        out_specs=pl.BlockSpec((B, D), lambda i: (i, 0)),
        out_shape=jax.ShapeDtypeStruct((E, D), jnp.float32),
    )(edge_attr, We)


def _hmm_body(x_ref, w_ref, b_ref, out_ref):
    out_ref[...] = jnp.dot(x_ref[...], w_ref[...],
                           preferred_element_type=jnp.float32) + b_ref[...]


def _hmm(x, W, b2):
    B = 2000
    return pl.pallas_call(
        _hmm_body,
        grid=(N // B,),
        in_specs=[pl.BlockSpec((B, D), lambda i: (i, 0)),
                  pl.BlockSpec((D, D), lambda i: (0, 0)),
                  pl.BlockSpec((1, D), lambda i: (0, 0))],
        out_specs=pl.BlockSpec((B, D), lambda i: (i, 0)),
        out_shape=jax.ShapeDtypeStruct((N, D), jnp.float32),
    )(x, W, b2)


def _upd_common(x_ref, h_ref, a0_ref, a1_ref, g_ref, be_ref, rm_ref, rv_ref):
    u = h_ref[...] + a0_ref[...] + a1_ref[...]
    scale = g_ref[...] * lax.rsqrt(rv_ref[...] + 1e-5)
    u = (u - rm_ref[...]) * scale + be_ref[...]
    u = jnp.maximum(u, 0.0)
    return 0.5 * x_ref[...] + 0.5 * u


def _updmm_body(x_ref, h_ref, a0_ref, a1_ref, g_ref, be_ref, rm_ref, rv_ref,
                w_ref, b_ref, xo_ref, ho_ref):
    xn = _upd_common(x_ref, h_ref, a0_ref, a1_ref, g_ref, be_ref, rm_ref, rv_ref)
    xo_ref[...] = xn
    ho_ref[...] = jnp.dot(xn, w_ref[...],
                          preferred_element_type=jnp.float32) + b_ref[...]


def _upd_last_body(x_ref, h_ref, a0_ref, a1_ref, g_ref, be_ref, rm_ref, rv_ref,
                   xo_ref):
    xo_ref[...] = _upd_common(x_ref, h_ref, a0_ref, a1_ref,
                              g_ref, be_ref, rm_ref, rv_ref)


def _update_mm(x, h, a0, a1, g2, be2, rm2, rv2, W, b2):
    B = 2000
    row = lambda i: (i, 0)
    fixed = lambda i: (0, 0)
    return pl.pallas_call(
        _updmm_body,
        grid=(N // B,),
        in_specs=[pl.BlockSpec((B, D), row), pl.BlockSpec((B, D), row),
                  pl.BlockSpec((B, D), row), pl.BlockSpec((B, D), row),
                  pl.BlockSpec((1, D), fixed), pl.BlockSpec((1, D), fixed),
                  pl.BlockSpec((1, D), fixed), pl.BlockSpec((1, D), fixed),
                  pl.BlockSpec((D, D), fixed), pl.BlockSpec((1, D), fixed)],
        out_specs=[pl.BlockSpec((B, D), row), pl.BlockSpec((B, D), row)],
        out_shape=[jax.ShapeDtypeStruct((N, D), jnp.float32),
                   jax.ShapeDtypeStruct((N, D), jnp.float32)],
    )(x, h, a0, a1, g2, be2, rm2, rv2, W, b2)


def _update_last(x, h, a0, a1, g2, be2, rm2, rv2):
    B = 2000
    row = lambda i: (i, 0)
    fixed = lambda i: (0, 0)
    return pl.pallas_call(
        _upd_last_body,
        grid=(N // B,),
        in_specs=[pl.BlockSpec((B, D), row), pl.BlockSpec((B, D), row),
                  pl.BlockSpec((B, D), row), pl.BlockSpec((B, D), row),
                  pl.BlockSpec((1, D), fixed), pl.BlockSpec((1, D), fixed),
                  pl.BlockSpec((1, D), fixed), pl.BlockSpec((1, D), fixed)],
        out_specs=pl.BlockSpec((B, D), row),
        out_shape=jax.ShapeDtypeStruct((N, D), jnp.float32),
    )(x, h, a0, a1, g2, be2, rm2, rv2)


def _edge_pass_body(h_hbm, src_hbm, dst_hbm, e_hbm, out_hbm,
                    srcv, dstv, hrows, erows, zbuf, agg_sh, sem_g, sem_e):
    c = lax.axis_index("c")
    s = lax.axis_index("s")
    wid = s * NC + c

    def zrow(j, _):
        for t in range(D // 16):
            zbuf[j, pl.ds(t * 16, 16)] = jnp.zeros((16,), jnp.float32)
        return 0
    lax.fori_loop(0, ZROWS, zrow, 0)
    for k in range(ROWS_PT // ZROWS):
        pltpu.sync_copy(zbuf, agg_sh.at[pl.ds(s * ROWS_PT + k * ZROWS, ZROWS)])
    plsc.subcore_barrier()

    def chunk(i, _):
        base = wid * EPW + i * CHUNK
        pltpu.sync_copy(src_hbm.at[pl.ds(base, CHUNK)], srcv)
        pltpu.sync_copy(dst_hbm.at[pl.ds(base, CHUNK)], dstv)
        cg = pltpu.async_copy(h_hbm.at[srcv], hrows, sem_g)
        ce = pltpu.async_copy(e_hbm.at[pl.ds(base, CHUNK)], erows, sem_e)
        cg.wait()
        ce.wait()

        def row(j, _):
            for t in range(D // 16):
                sl = pl.ds(t * 16, 16)
                hrows[j, sl] = jnp.maximum(hrows[j, sl] + erows[j, sl], 0.0)
            return 0
        lax.fori_loop(0, CHUNK, row, 0)
        pltpu.sync_copy(hrows, agg_sh.at[dstv], add=True)
        return 0
    lax.fori_loop(0, NCHUNK, chunk, 0)
    plsc.subcore_barrier()

    pltpu.sync_copy(agg_sh.at[pl.ds(s * ROWS_PT, ROWS_PT)],
                    out_hbm.at[c, pl.ds(s * ROWS_PT, ROWS_PT)])


_edge_pass = functools.partial(
    pl.kernel,
    out_type=jax.ShapeDtypeStruct((NC, NPAD, D), jnp.float32),
    mesh=plsc.VectorSubcoreMesh(core_axis_name="c", subcore_axis_name="s"),
    scratch_types=[
        pltpu.VMEM((CHUNK,), jnp.int32),
        pltpu.VMEM((CHUNK,), jnp.int32),
        pltpu.VMEM((CHUNK, D), jnp.float32),
        pltpu.VMEM((CHUNK, D), jnp.float32),
        pltpu.VMEM((ZROWS, D), jnp.float32),
        pltpu.VMEM_SHARED((NPAD, D), jnp.float32),
        pltpu.SemaphoreType.DMA,
        pltpu.SemaphoreType.DMA,
    ],
)(_edge_pass_body)


def kernel(x, edge_index, edge_attr, batch, W, b, We, gamma, beta,
           run_mean, run_var):
    src = edge_index[0]
    dst = edge_index[1]
    b2 = b.reshape(1, D)
    g2 = gamma.reshape(1, D)
    be2 = beta.reshape(1, D)
    rm2 = run_mean.reshape(1, D)
    rv2 = run_var.reshape(1, D)

    e = _edge_feat(edge_attr, We)
    h = _hmm(x, W, b2)
    for i in range(4):
        aggs = _edge_pass(h, src, dst, e)
        a0 = aggs[0, :N]
        a1 = aggs[1, :N]
        if i < 3:
            x, h = _update_mm(x, h, a0, a1, g2, be2, rm2, rv2, W, b2)
        else:
            x = _update_last(x, h, a0, a1, g2, be2, rm2, rv2)
    return x


# speedup vs baseline: 1.9600x; 1.2631x over previous
import functools

import jax
import jax.numpy as jnp
from jax import lax
from jax.experimental import pallas as pl
from jax.experimental.pallas import tpu as pltpu
from jax.experimental.pallas import tpu_sc as plsc

N = 10000
E = 320000
D = 128
DE = 4

NC = 2
NS = 16
NW = NC * NS
CHUNK = 80
EPW = E // NW
NCHUNK = EPW // CHUNK
NPAD = 10240
ROWS_PT = NPAD // NS
ZROWS = 128


def _ef_body(ea_ref, we_ref, out_ref):
    out_ref[...] = jnp.dot(ea_ref[...], we_ref[...],
                           preferred_element_type=jnp.float32)


def _edge_feat(edge_attr, We):
    B = 4000
    return pl.pallas_call(
        _ef_body,
        grid=(E // B,),
        in_specs=[pl.BlockSpec((B, DE), lambda i: (i, 0)),
                  pl.BlockSpec((DE, D), lambda i: (0, 0))],
        out_specs=pl.BlockSpec((B, D), lambda i: (i, 0)),
        out_shape=jax.ShapeDtypeStruct((E, D), jnp.float32),
    )(edge_attr, We)


def _hmm_body(x_ref, w_ref, b_ref, out_ref):
    out_ref[...] = jnp.dot(x_ref[...], w_ref[...],
                           preferred_element_type=jnp.float32) + b_ref[...]


def _hmm(x, W, b2):
    B = 2000
    return pl.pallas_call(
        _hmm_body,
        grid=(N // B,),
        in_specs=[pl.BlockSpec((B, D), lambda i: (i, 0)),
                  pl.BlockSpec((D, D), lambda i: (0, 0)),
                  pl.BlockSpec((1, D), lambda i: (0, 0))],
        out_specs=pl.BlockSpec((B, D), lambda i: (i, 0)),
        out_shape=jax.ShapeDtypeStruct((N, D), jnp.float32),
    )(x, W, b2)


def _upd_common(x_ref, h_ref, a0_ref, a1_ref, g_ref, be_ref, rm_ref, rv_ref):
    u = h_ref[...] + a0_ref[...] + a1_ref[...]
    scale = g_ref[...] * lax.rsqrt(rv_ref[...] + 1e-5)
    u = (u - rm_ref[...]) * scale + be_ref[...]
    u = jnp.maximum(u, 0.0)
    return 0.5 * x_ref[...] + 0.5 * u


def _updmm_body(x_ref, h_ref, a0_ref, a1_ref, g_ref, be_ref, rm_ref, rv_ref,
                w_ref, b_ref, xo_ref, ho_ref):
    xn = _upd_common(x_ref, h_ref, a0_ref, a1_ref, g_ref, be_ref, rm_ref, rv_ref)
    xo_ref[...] = xn
    ho_ref[...] = jnp.dot(xn, w_ref[...],
                          preferred_element_type=jnp.float32) + b_ref[...]


def _upd_last_body(x_ref, h_ref, a0_ref, a1_ref, g_ref, be_ref, rm_ref, rv_ref,
                   xo_ref):
    xo_ref[...] = _upd_common(x_ref, h_ref, a0_ref, a1_ref,
                              g_ref, be_ref, rm_ref, rv_ref)


def _update_mm(x, h, a0, a1, g2, be2, rm2, rv2, W, b2):
    B = 2000
    row = lambda i: (i, 0)
    fixed = lambda i: (0, 0)
    return pl.pallas_call(
        _updmm_body,
        grid=(N // B,),
        in_specs=[pl.BlockSpec((B, D), row), pl.BlockSpec((B, D), row),
                  pl.BlockSpec((B, D), row), pl.BlockSpec((B, D), row),
                  pl.BlockSpec((1, D), fixed), pl.BlockSpec((1, D), fixed),
                  pl.BlockSpec((1, D), fixed), pl.BlockSpec((1, D), fixed),
                  pl.BlockSpec((D, D), fixed), pl.BlockSpec((1, D), fixed)],
        out_specs=[pl.BlockSpec((B, D), row), pl.BlockSpec((B, D), row)],
        out_shape=[jax.ShapeDtypeStruct((N, D), jnp.float32),
                   jax.ShapeDtypeStruct((N, D), jnp.float32)],
    )(x, h, a0, a1, g2, be2, rm2, rv2, W, b2)


def _update_last(x, h, a0, a1, g2, be2, rm2, rv2):
    B = 2000
    row = lambda i: (i, 0)
    fixed = lambda i: (0, 0)
    return pl.pallas_call(
        _upd_last_body,
        grid=(N // B,),
        in_specs=[pl.BlockSpec((B, D), row), pl.BlockSpec((B, D), row),
                  pl.BlockSpec((B, D), row), pl.BlockSpec((B, D), row),
                  pl.BlockSpec((1, D), fixed), pl.BlockSpec((1, D), fixed),
                  pl.BlockSpec((1, D), fixed), pl.BlockSpec((1, D), fixed)],
        out_specs=pl.BlockSpec((B, D), row),
        out_shape=jax.ShapeDtypeStruct((N, D), jnp.float32),
    )(x, h, a0, a1, g2, be2, rm2, rv2)


def _edge_pass_body(h_hbm, src_hbm, dst_hbm, e_hbm, out_hbm,
                    sv0, sv1, dv0, dv1, hrows, erows, zbuf, agg_sh,
                    sem_g, sem_e, sem_is0, sem_is1, sem_id0, sem_id1):
    c = lax.axis_index("c")
    s = lax.axis_index("s")
    wid = s * NC + c
    sv = (sv0, sv1)
    dv = (dv0, dv1)
    sem_is = (sem_is0, sem_is1)
    sem_id = (sem_id0, sem_id1)

    def zrow(j, _):
        for t in range(D // 16):
            zbuf[j, pl.ds(t * 16, 16)] = jnp.zeros((16,), jnp.float32)
        return 0
    lax.fori_loop(0, ZROWS, zrow, 0)
    for k in range(ROWS_PT // ZROWS):
        pltpu.sync_copy(zbuf, agg_sh.at[pl.ds(s * ROWS_PT + k * ZROWS, ZROWS)])
    plsc.subcore_barrier()

    def issue_idx(i, p):
        base = wid * EPW + i * CHUNK
        pltpu.async_copy(src_hbm.at[pl.ds(base, CHUNK)], sv[p], sem_is[p])
        pltpu.async_copy(dst_hbm.at[pl.ds(base, CHUNK)], dv[p], sem_id[p])

    def wait_idx(p):
        pltpu.make_async_copy(src_hbm.at[pl.ds(0, CHUNK)], sv[p],
                              sem_is[p]).wait()
        pltpu.make_async_copy(dst_hbm.at[pl.ds(0, CHUNK)], dv[p],
                              sem_id[p]).wait()

    def chunk(i, p, has_next):
        base = wid * EPW + i * CHUNK
        wait_idx(p)
        if has_next:
            issue_idx(i + 1, 1 - p)
        cg = pltpu.async_copy(h_hbm.at[sv[p]], hrows, sem_g)
        ce = pltpu.async_copy(e_hbm.at[pl.ds(base, CHUNK)], erows, sem_e)
        cg.wait()
        ce.wait()

        def row(j, _):
            for t in range(D // 16):
                sl = pl.ds(t * 16, 16)
                hrows[j, sl] = jnp.maximum(hrows[j, sl] + erows[j, sl], 0.0)
            return 0
        lax.fori_loop(0, CHUNK, row, 0)
        pltpu.sync_copy(hrows, agg_sh.at[dv[p]], add=True)

    issue_idx(0, 0)

    def pairbody(k, _):
        chunk(2 * k, 0, True)
        chunk(2 * k + 1, 1, True)
        return 0
    lax.fori_loop(0, NCHUNK // 2, pairbody, 0)
    chunk(NCHUNK - 1, 0, False)
    plsc.subcore_barrier()

    pltpu.sync_copy(agg_sh.at[pl.ds(s * ROWS_PT, ROWS_PT)],
                    out_hbm.at[c, pl.ds(s * ROWS_PT, ROWS_PT)])


_edge_pass = functools.partial(
    pl.kernel,
    out_type=jax.ShapeDtypeStruct((NC, NPAD, D), jnp.float32),
    mesh=plsc.VectorSubcoreMesh(core_axis_name="c", subcore_axis_name="s"),
    scratch_types=[
        pltpu.VMEM((CHUNK,), jnp.int32),
        pltpu.VMEM((CHUNK,), jnp.int32),
        pltpu.VMEM((CHUNK,), jnp.int32),
        pltpu.VMEM((CHUNK,), jnp.int32),
        pltpu.VMEM((CHUNK, D), jnp.float32),
        pltpu.VMEM((CHUNK, D), jnp.float32),
        pltpu.VMEM((ZROWS, D), jnp.float32),
        pltpu.VMEM_SHARED((NPAD, D), jnp.float32),
        pltpu.SemaphoreType.DMA,
        pltpu.SemaphoreType.DMA,
        pltpu.SemaphoreType.DMA,
        pltpu.SemaphoreType.DMA,
        pltpu.SemaphoreType.DMA,
        pltpu.SemaphoreType.DMA,
    ],
)(_edge_pass_body)


def kernel(x, edge_index, edge_attr, batch, W, b, We, gamma, beta,
           run_mean, run_var):
    src = edge_index[0]
    dst = edge_index[1]
    b2 = b.reshape(1, D)
    g2 = gamma.reshape(1, D)
    be2 = beta.reshape(1, D)
    rm2 = run_mean.reshape(1, D)
    rv2 = run_var.reshape(1, D)

    e = _edge_feat(edge_attr, We)
    h = _hmm(x, W, b2)
    for i in range(4):
        aggs = _edge_pass(h, src, dst, e)
        a0 = aggs[0, :N]
        a1 = aggs[1, :N]
        if i < 3:
            x, h = _update_mm(x, h, a0, a1, g2, be2, rm2, rv2, W, b2)
        else:
            x = _update_last(x, h, a0, a1, g2, be2, rm2, rv2)
    return x


# speedup vs baseline: 1.9642x; 1.0021x over previous
import functools

import jax
import jax.numpy as jnp
from jax import lax
from jax.experimental import pallas as pl
from jax.experimental.pallas import tpu as pltpu
from jax.experimental.pallas import tpu_sc as plsc

N = 10000
E = 320000
D = 128
DE = 4

NC = 2
NS = 16
NW = NC * NS
CHUNK = 80
EPW = E // NW
NCHUNK = EPW // CHUNK
NPAD = 10240
ROWS_PT = NPAD // NS
ZROWS = 8


def _ef_body(ea_ref, we_ref, out_ref):
    out_ref[...] = jnp.dot(ea_ref[...], we_ref[...],
                           preferred_element_type=jnp.float32)


def _edge_feat(edge_attr, We):
    B = 4000
    return pl.pallas_call(
        _ef_body,
        grid=(E // B,),
        in_specs=[pl.BlockSpec((B, DE), lambda i: (i, 0)),
                  pl.BlockSpec((DE, D), lambda i: (0, 0))],
        out_specs=pl.BlockSpec((B, D), lambda i: (i, 0)),
        out_shape=jax.ShapeDtypeStruct((E, D), jnp.float32),
    )(edge_attr, We)


def _hmm_body(x_ref, w_ref, b_ref, out_ref):
    out_ref[...] = jnp.dot(x_ref[...], w_ref[...],
                           preferred_element_type=jnp.float32) + b_ref[...]


def _hmm(x, W, b2):
    B = 2000
    return pl.pallas_call(
        _hmm_body,
        grid=(N // B,),
        in_specs=[pl.BlockSpec((B, D), lambda i: (i, 0)),
                  pl.BlockSpec((D, D), lambda i: (0, 0)),
                  pl.BlockSpec((1, D), lambda i: (0, 0))],
        out_specs=pl.BlockSpec((B, D), lambda i: (i, 0)),
        out_shape=jax.ShapeDtypeStruct((N, D), jnp.float32),
    )(x, W, b2)


def _upd_common(x_ref, h_ref, a0_ref, a1_ref, g_ref, be_ref, rm_ref, rv_ref):
    u = h_ref[...] + a0_ref[...] + a1_ref[...]
    scale = g_ref[...] * lax.rsqrt(rv_ref[...] + 1e-5)
    u = (u - rm_ref[...]) * scale + be_ref[...]
    u = jnp.maximum(u, 0.0)
    return 0.5 * x_ref[...] + 0.5 * u


def _updmm_body(x_ref, h_ref, a0_ref, a1_ref, g_ref, be_ref, rm_ref, rv_ref,
                w_ref, b_ref, xo_ref, ho_ref):
    xn = _upd_common(x_ref, h_ref, a0_ref, a1_ref, g_ref, be_ref, rm_ref, rv_ref)
    xo_ref[...] = xn
    ho_ref[...] = jnp.dot(xn, w_ref[...],
                          preferred_element_type=jnp.float32) + b_ref[...]


def _upd_last_body(x_ref, h_ref, a0_ref, a1_ref, g_ref, be_ref, rm_ref, rv_ref,
                   xo_ref):
    xo_ref[...] = _upd_common(x_ref, h_ref, a0_ref, a1_ref,
                              g_ref, be_ref, rm_ref, rv_ref)


def _update_mm(x, h, a0, a1, g2, be2, rm2, rv2, W, b2):
    B = 2000
    row = lambda i: (i, 0)
    fixed = lambda i: (0, 0)
    return pl.pallas_call(
        _updmm_body,
        grid=(N // B,),
        in_specs=[pl.BlockSpec((B, D), row), pl.BlockSpec((B, D), row),
                  pl.BlockSpec((B, D), row), pl.BlockSpec((B, D), row),
                  pl.BlockSpec((1, D), fixed), pl.BlockSpec((1, D), fixed),
                  pl.BlockSpec((1, D), fixed), pl.BlockSpec((1, D), fixed),
                  pl.BlockSpec((D, D), fixed), pl.BlockSpec((1, D), fixed)],
        out_specs=[pl.BlockSpec((B, D), row), pl.BlockSpec((B, D), row)],
        out_shape=[jax.ShapeDtypeStruct((N, D), jnp.float32),
                   jax.ShapeDtypeStruct((N, D), jnp.float32)],
    )(x, h, a0, a1, g2, be2, rm2, rv2, W, b2)


def _update_last(x, h, a0, a1, g2, be2, rm2, rv2):
    B = 2000
    row = lambda i: (i, 0)
    fixed = lambda i: (0, 0)
    return pl.pallas_call(
        _upd_last_body,
        grid=(N // B,),
        in_specs=[pl.BlockSpec((B, D), row), pl.BlockSpec((B, D), row),
                  pl.BlockSpec((B, D), row), pl.BlockSpec((B, D), row),
                  pl.BlockSpec((1, D), fixed), pl.BlockSpec((1, D), fixed),
                  pl.BlockSpec((1, D), fixed), pl.BlockSpec((1, D), fixed)],
        out_specs=pl.BlockSpec((B, D), row),
        out_shape=jax.ShapeDtypeStruct((N, D), jnp.float32),
    )(x, h, a0, a1, g2, be2, rm2, rv2)


def _edge_pass_body(h_hbm, src_hbm, dst_hbm, e_hbm, out_hbm,
                    sv0, sv1, dv0, dv1, hb0, hb1, erows, zbuf, agg_sh,
                    sem_g0, sem_g1, sem_e, sem_is0, sem_is1, sem_id0, sem_id1):
    c = lax.axis_index("c")
    s = lax.axis_index("s")
    wid = s * NC + c
    sv = (sv0, sv1)
    dv = (dv0, dv1)
    hb = (hb0, hb1)
    sem_g = (sem_g0, sem_g1)
    sem_is = (sem_is0, sem_is1)
    sem_id = (sem_id0, sem_id1)

    def zrow(j, _):
        for t in range(D // 16):
            zbuf[j, pl.ds(t * 16, 16)] = jnp.zeros((16,), jnp.float32)
        return 0
    lax.fori_loop(0, ZROWS, zrow, 0)
    for k in range(ROWS_PT // ZROWS):
        pltpu.sync_copy(zbuf, agg_sh.at[pl.ds(s * ROWS_PT + k * ZROWS, ZROWS)])
    plsc.subcore_barrier()

    def issue_src(i, p):
        pltpu.async_copy(src_hbm.at[pl.ds(wid * EPW + i * CHUNK, CHUNK)],
                         sv[p], sem_is[p])

    def wait_src(p):
        pltpu.make_async_copy(src_hbm.at[pl.ds(0, CHUNK)], sv[p],
                              sem_is[p]).wait()

    def issue_dst(i, p):
        pltpu.async_copy(dst_hbm.at[pl.ds(wid * EPW + i * CHUNK, CHUNK)],
                         dv[p], sem_id[p])

    def wait_dst(p):
        pltpu.make_async_copy(dst_hbm.at[pl.ds(0, CHUNK)], dv[p],
                              sem_id[p]).wait()

    def issue_g(i, p):
        pltpu.async_copy(h_hbm.at[sv[p]], hb[p], sem_g[p])

    def wait_g(p):
        pltpu.make_async_copy(h_hbm.at[sv[p]], hb[p], sem_g[p]).wait()

    def chunk(i, p, next_g, next_idx):
        if next_g:
            wait_src(1 - p)
            issue_g(i + 1, 1 - p)
        ce = pltpu.async_copy(e_hbm.at[pl.ds(wid * EPW + i * CHUNK, CHUNK)],
                              erows, sem_e)
        wait_g(p)
        if next_idx:
            issue_src(i + 2, p)
        ce.wait()
        wait_dst(p)

        def row(j, _):
            for t in range(D // 16):
                sl = pl.ds(t * 16, 16)
                hb[p][j, sl] = jnp.maximum(hb[p][j, sl] + erows[j, sl], 0.0)
            return 0
        lax.fori_loop(0, CHUNK, row, 0)
        pltpu.sync_copy(hb[p], agg_sh.at[dv[p]], add=True)
        if next_idx:
            issue_dst(i + 2, p)

    issue_src(0, 0)
    issue_dst(0, 0)
    wait_src(0)
    issue_g(0, 0)
    issue_src(1, 1)
    issue_dst(1, 1)

    def pairbody(k, _):
        chunk(2 * k, 0, True, True)
        chunk(2 * k + 1, 1, True, True)
        return 0
    lax.fori_loop(0, (NCHUNK - 3) // 2, pairbody, 0)
    chunk(NCHUNK - 3, 0, True, True)
    chunk(NCHUNK - 2, 1, True, False)
    chunk(NCHUNK - 1, 0, False, False)
    plsc.subcore_barrier()

    pltpu.sync_copy(agg_sh.at[pl.ds(s * ROWS_PT, ROWS_PT)],
                    out_hbm.at[c, pl.ds(s * ROWS_PT, ROWS_PT)])


_edge_pass = functools.partial(
    pl.kernel,
    out_type=jax.ShapeDtypeStruct((NC, NPAD, D), jnp.float32),
    mesh=plsc.VectorSubcoreMesh(core_axis_name="c", subcore_axis_name="s"),
    scratch_types=[
        pltpu.VMEM((CHUNK,), jnp.int32),
        pltpu.VMEM((CHUNK,), jnp.int32),
        pltpu.VMEM((CHUNK,), jnp.int32),
        pltpu.VMEM((CHUNK,), jnp.int32),
        pltpu.VMEM((CHUNK, D), jnp.float32),
        pltpu.VMEM((CHUNK, D), jnp.float32),
        pltpu.VMEM((CHUNK, D), jnp.float32),
        pltpu.VMEM((ZROWS, D), jnp.float32),
        pltpu.VMEM_SHARED((NPAD, D), jnp.float32),
        pltpu.SemaphoreType.DMA,
        pltpu.SemaphoreType.DMA,
        pltpu.SemaphoreType.DMA,
        pltpu.SemaphoreType.DMA,
        pltpu.SemaphoreType.DMA,
        pltpu.SemaphoreType.DMA,
        pltpu.SemaphoreType.DMA,
    ],
)(_edge_pass_body)


def kernel(x, edge_index, edge_attr, batch, W, b, We, gamma, beta,
           run_mean, run_var):
    src = edge_index[0]
    dst = edge_index[1]
    b2 = b.reshape(1, D)
    g2 = gamma.reshape(1, D)
    be2 = beta.reshape(1, D)
    rm2 = run_mean.reshape(1, D)
    rv2 = run_var.reshape(1, D)

    e = _edge_feat(edge_attr, We)
    h = _hmm(x, W, b2)
    for i in range(4):
        aggs = _edge_pass(h, src, dst, e)
        a0 = aggs[0, :N]
        a1 = aggs[1, :N]
        if i < 3:
            x, h = _update_mm(x, h, a0, a1, g2, be2, rm2, rv2, W, b2)
        else:
            x = _update_last(x, h, a0, a1, g2, be2, rm2, rv2)
    return x
